# baseline (speedup 1.0000x reference)
import jax
import jax.numpy as jnp
from jax.experimental import pallas as pl
from jax.experimental.pallas import tpu as pltpu


def _gather_body(idx_ref, table_ref, out_ref, *, rows_per_block):
    base = pl.program_id(0) * rows_per_block
    for r in range(rows_per_block):
        out_ref[r] = table_ref[idx_ref[base + r]]


def kernel(prefix, table):
    b, s = prefix.shape
    vocab, width = table.shape
    n = b * s
    lanes = 128
    sub = width // lanes
    rows_per_block = 32

    idx = prefix.reshape(n).astype(jnp.int32)
    table3 = table.reshape(vocab, sub, lanes)

    import functools
    body = functools.partial(_gather_body, rows_per_block=rows_per_block)

    grid_spec = pltpu.PrefetchScalarGridSpec(
        num_scalar_prefetch=1,
        grid=(n // rows_per_block,),
        in_specs=[
            pl.BlockSpec((vocab, sub, lanes), lambda i, idx_ref: (0, 0, 0)),
        ],
        out_specs=pl.BlockSpec((rows_per_block, sub, lanes),
                               lambda i, idx_ref: (i, 0, 0)),
    )
    out = pl.pallas_call(
        body,
        grid_spec=grid_spec,
        out_shape=jax.ShapeDtypeStruct((n, sub, lanes), table.dtype),
    )(idx, table3)
    return out.reshape(b, s, width)


# speedup vs baseline: 1.7325x; 1.7325x over previous
import functools

import jax
import jax.numpy as jnp
from jax import lax
from jax.experimental import pallas as pl
from jax.experimental.pallas import tpu as pltpu
from jax.experimental.pallas import tpu_sc as plsc

_NC = 2
_NS = 16
_NW = _NC * _NS
_CHUNK = 4
_NBUF = 2


def _sc_body(table_hbm, idx_hbm, out_hbm, idx_v, bufs, sem_g, sem_s,
             *, nchunks_per_w, width):
    wid = lax.axis_index("s") * _NC + lax.axis_index("c")
    base = wid * nchunks_per_w
    pltpu.sync_copy(idx_hbm.at[pl.ds(base, nchunks_per_w)], idx_v)

    def gather_start(j, b):
        pltpu.async_copy(table_hbm.at[idx_v.at[j]], bufs.at[b], sem_g.at[b])

    def gather_wait(b):
        pltpu.make_async_copy(
            table_hbm.at[idx_v.at[0]], bufs.at[b], sem_g.at[b]).wait()

    def scatter_start(j, b):
        pltpu.async_copy(
            bufs.at[b], out_hbm.at[pl.ds((base + j) * _CHUNK, _CHUNK)],
            sem_s.at[b])

    def scatter_wait(b):
        pltpu.make_async_copy(
            bufs.at[b], out_hbm.at[pl.ds(0, _CHUNK)], sem_s.at[b]).wait()

    for b in range(_NBUF):
        gather_start(b, b)

    @pl.loop(0, nchunks_per_w, step=_NBUF)
    def _pipeline(jj):
        for b in range(_NBUF):
            j = jj + b
            gather_wait(b)
            scatter_start(j, b)

            @pl.when(j + _NBUF < nchunks_per_w)
            def _refill():
                scatter_wait(b)
                gather_start(j + _NBUF, b)

    for b in range(_NBUF):
        scatter_wait(b)


def kernel(prefix, table):
    bsz, seq = prefix.shape
    n = bsz * seq
    vocab, width = table.shape
    nchunks_per_w = n // (_NW * _CHUNK)

    idx2 = prefix.reshape(n // _CHUNK, _CHUNK).astype(jnp.int32)
    mesh = plsc.VectorSubcoreMesh(core_axis_name="c", subcore_axis_name="s")
    body = functools.partial(_sc_body, nchunks_per_w=nchunks_per_w,
                             width=width)
    k = pl.kernel(
        body,
        out_type=jax.ShapeDtypeStruct((n, width), table.dtype),
        mesh=mesh,
        scratch_types=[
            pltpu.VMEM((nchunks_per_w, _CHUNK), jnp.int32),
            pltpu.VMEM((_NBUF, _CHUNK, width), table.dtype),
            pltpu.SemaphoreType.DMA((_NBUF,)),
            pltpu.SemaphoreType.DMA((_NBUF,)),
        ],
    )
    out = k(table, idx2)
    return out.reshape(bsz, seq, width)
